# Initial kernel scaffold; baseline (speedup 1.0000x reference)
import jax
import jax.numpy as jnp
from jax.experimental import pallas as pl


def kernel(x, original_edge_index, dg_edge_index, batch, params):
    raise NotImplementedError("write your pallas kernel here")


# speedup vs baseline: 9.4731x; 9.4731x over previous
import functools

import jax
import jax.numpy as jnp
from jax import lax
from jax.experimental import pallas as pl
from jax.experimental.pallas import tpu as pltpu
from jax.experimental.pallas import tpu_sc as plsc

NSC = 2
NTILE = 16
LANE = 128
BN = 1000


def _mesh():
    return plsc.VectorSubcoreMesh(core_axis_name="c", subcore_axis_name="s")


def _sc_degree(dstb_rows, zeros16, ones16, n):
    _, R, _ = dstb_rows.shape
    npad = -(-n // LANE) * LANE
    slab = npad // NTILE
    iters = (R + NTILE - 1) // NTILE

    @functools.partial(
        pl.kernel,
        out_type=jax.ShapeDtypeStruct((2, npad, 16), jnp.float32),
        mesh=_mesh(),
        scratch_types=[
            pltpu.VMEM((LANE,), jnp.int32),
            pltpu.VMEM((LANE, 16), jnp.float32),
            pltpu.VMEM_SHARED((npad, 16), jnp.float32),
            pltpu.SemaphoreType.DMA,
        ],
    )
    def k(dstb_hbm, zeros_hbm, ones_hbm, out_hbm, dst_v, ones_v, acc_sh, sem):
        c = lax.axis_index("c")
        s = lax.axis_index("s")
        pltpu.sync_copy(ones_hbm, ones_v)
        pltpu.sync_copy(zeros_hbm, acc_sh.at[pl.ds(s * slab, slab)])
        plsc.subcore_barrier()

        def body(i, carry):
            r = s + i * NTILE
            @pl.when(r < R)
            def _():
                pltpu.sync_copy(dstb_hbm.at[c, r], dst_v)
                pltpu.sync_copy(ones_v, acc_sh.at[dst_v], add=True)
            return carry

        lax.fori_loop(0, iters, body, 0)
        plsc.subcore_barrier()
        pltpu.sync_copy(acc_sh.at[pl.ds(s * slab, slab)],
                        out_hbm.at[c, pl.ds(s * slab, slab)])

    return k(dstb_rows, zeros16, ones16)


def _sc_conv(table2, srcb_rows, dst_rows, zeros128, n):
    R = dst_rows.shape[0]
    npad = -(-n // LANE) * LANE
    slab = npad // NTILE
    iters = (R + NTILE - 1) // NTILE

    @functools.partial(
        pl.kernel,
        out_type=jax.ShapeDtypeStruct((2, npad, 128), jnp.float32),
        mesh=_mesh(),
        scratch_types=[
            pltpu.VMEM((LANE,), jnp.int32),
            pltpu.VMEM((LANE,), jnp.int32),
            pltpu.VMEM((LANE, 128), jnp.float32),
            pltpu.VMEM_SHARED((npad, 128), jnp.float32),
            pltpu.SemaphoreType.DMA,
        ],
    )
    def k(table_hbm, srcb_hbm, dstr_hbm, zeros_hbm, out_hbm,
          src_v, dst_v, rows_v, acc_sh, sem):
        c = lax.axis_index("c")
        s = lax.axis_index("s")
        pltpu.sync_copy(zeros_hbm, acc_sh.at[pl.ds(s * slab, slab)])
        plsc.subcore_barrier()

        def body(i, carry):
            r = s + i * NTILE
            @pl.when(r < R)
            def _():
                pltpu.sync_copy(srcb_hbm.at[c, r], src_v)
                pltpu.sync_copy(dstr_hbm.at[r], dst_v)
                pltpu.async_copy(table_hbm.at[src_v], rows_v, sem).wait()
                pltpu.sync_copy(rows_v, acc_sh.at[dst_v], add=True)
            return carry

        lax.fori_loop(0, iters, body, 0)
        plsc.subcore_barrier()
        pltpu.sync_copy(acc_sh.at[pl.ds(s * slab, slab)],
                        out_hbm.at[c, pl.ds(s * slab, slab)])

    return k(table2, srcb_rows, dst_rows, zeros128)


def _tc_prep(x, hist, W1, b1):
    n, d_in = x.shape
    dh = W1.shape[1]
    nb = n // BN

    def body(x_ref, h_ref, w_ref, b_ref, dinv_ref, hp_ref):
        deg = h_ref[:, 0:1] + 1.0
        dinv = lax.rsqrt(deg)
        dinv_ref[...] = jnp.broadcast_to(dinv, (BN, 128))
        t = jnp.dot(x_ref[...], w_ref[...], preferred_element_type=jnp.float32)
        t = (t + b_ref[...]) * dinv
        hp_ref[0] = t[:, :128]
        hp_ref[1] = t[:, 128:]

    return pl.pallas_call(
        body,
        grid=(nb,),
        in_specs=[
            pl.BlockSpec((BN, d_in), lambda i: (i, 0)),
            pl.BlockSpec((BN, 16), lambda i: (i, 0)),
            pl.BlockSpec((d_in, dh), lambda i: (0, 0)),
            pl.BlockSpec((1, dh), lambda i: (0, 0)),
        ],
        out_specs=[
            pl.BlockSpec((BN, 128), lambda i: (i, 0)),
            pl.BlockSpec((2, BN, 128), lambda i: (0, i, 0)),
        ],
        out_shape=[
            jax.ShapeDtypeStruct((n, 128), jnp.float32),
            jax.ShapeDtypeStruct((2, n, 128), jnp.float32),
        ],
    )(x, hist, W1, b1.reshape(1, dh))


def _tc_mid(asum, hp, dinv, W2, b2):
    n = dinv.shape[0]
    dh = W2.shape[0]
    nb = n // BN

    def body(a_ref, p_ref, d_ref, w_ref, b_ref, out_ref):
        di = d_ref[...]
        h0 = jnp.maximum((a_ref[0] + p_ref[0]) * di, 0.0)
        h1 = jnp.maximum((a_ref[1] + p_ref[1]) * di, 0.0)
        t = jnp.dot(h0, w_ref[0:128, :], preferred_element_type=jnp.float32)
        t += jnp.dot(h1, w_ref[128:256, :], preferred_element_type=jnp.float32)
        t = (t + b_ref[...]) * di[:, 0:1]
        out_ref[0] = t[:, :128]
        out_ref[1] = t[:, 128:]

    return pl.pallas_call(
        body,
        grid=(nb,),
        in_specs=[
            pl.BlockSpec((2, BN, 128), lambda i: (0, i, 0)),
            pl.BlockSpec((2, BN, 128), lambda i: (0, i, 0)),
            pl.BlockSpec((BN, 128), lambda i: (i, 0)),
            pl.BlockSpec((dh, dh), lambda i: (0, 0)),
            pl.BlockSpec((1, dh), lambda i: (0, 0)),
        ],
        out_specs=pl.BlockSpec((2, BN, 128), lambda i: (0, i, 0)),
        out_shape=jax.ShapeDtypeStruct((2, n, 128), jnp.float32),
    )(asum, hp, dinv, W2, b2.reshape(1, dh))


def _tc_fuse1(asum_o, hp_o, dinv_o, asum_d, hp_d, dinv_d, batch3, nng, nnb, P):
    n = dinv_o.shape[0]
    nb = n // BN

    def body(ao, po, do_, ad, pd, dd, b3, g_ref, b_ref,
             orig_ref, dg_ref, xn_ref, sums_ref, cnt_ref):
        i = pl.program_id(0)
        dio = do_[...]
        orig = jnp.concatenate([(ao[0] + po[0]) * dio, (ao[1] + po[1]) * dio], axis=1)
        did = dd[...]
        dg = jnp.concatenate([(ad[0] + pd[0]) * did, (ad[1] + pd[1]) * did], axis=1)
        orig_ref[...] = orig
        dg_ref[...] = dg
        mu = jnp.mean(orig, axis=1, keepdims=True)
        var = jnp.mean((orig - mu) ** 2, axis=1, keepdims=True)
        xn = g_ref[...] * (orig - mu) * lax.rsqrt(var + 1e-5) + b_ref[...]
        xn_ref[...] = xn
        pid = lax.broadcasted_iota(jnp.int32, (P, BN), 0)
        ohT = (pid == jnp.broadcast_to(b3[0], (P, BN))).astype(jnp.float32)
        ps = jnp.dot(ohT, dg, preferred_element_type=jnp.float32)
        pc = jnp.sum(ohT, axis=1, keepdims=True)

        @pl.when(i == 0)
        def _():
            sums_ref[...] = jnp.zeros_like(sums_ref)
            cnt_ref[...] = jnp.zeros_like(cnt_ref)

        sums_ref[...] += ps
        cnt_ref[...] += jnp.broadcast_to(pc, (P, 128))

    return pl.pallas_call(
        body,
        grid=(nb,),
        in_specs=[
            pl.BlockSpec((2, BN, 128), lambda i: (0, i, 0)),
            pl.BlockSpec((2, BN, 128), lambda i: (0, i, 0)),
            pl.BlockSpec((BN, 128), lambda i: (i, 0)),
            pl.BlockSpec((2, BN, 128), lambda i: (0, i, 0)),
            pl.BlockSpec((2, BN, 128), lambda i: (0, i, 0)),
            pl.BlockSpec((BN, 128), lambda i: (i, 0)),
            pl.BlockSpec((1, 1, BN), lambda i: (i, 0, 0)),
            pl.BlockSpec((1, 256), lambda i: (0, 0)),
            pl.BlockSpec((1, 256), lambda i: (0, 0)),
        ],
        out_specs=[
            pl.BlockSpec((BN, 256), lambda i: (i, 0)),
            pl.BlockSpec((BN, 256), lambda i: (i, 0)),
            pl.BlockSpec((BN, 256), lambda i: (i, 0)),
            pl.BlockSpec((P, 256), lambda i: (0, 0)),
            pl.BlockSpec((P, 128), lambda i: (0, 0)),
        ],
        out_shape=[
            jax.ShapeDtypeStruct((n, 256), jnp.float32),
            jax.ShapeDtypeStruct((n, 256), jnp.float32),
            jax.ShapeDtypeStruct((n, 256), jnp.float32),
            jax.ShapeDtypeStruct((P, 256), jnp.float32),
            jax.ShapeDtypeStruct((P, 128), jnp.float32),
        ],
    )(asum_o, hp_o, dinv_o, asum_d, hp_d, dinv_d, batch3,
      nng.reshape(1, 256), nnb.reshape(1, 256))


def _tc_fuse2(sums, counts, p):
    P = sums.shape[0]

    def ln(h, g, b):
        mu = jnp.mean(h, axis=1, keepdims=True)
        var = jnp.mean((h - mu) ** 2, axis=1, keepdims=True)
        return g * (h - mu) * lax.rsqrt(var + 1e-5) + b

    def body(s_ref, c_ref, dgg, dgb, wv, bv, wo, bo, fng, fnb,
             w1, b1, w2, b2, whi, fb, gp_ref):
        cnt = jnp.maximum(c_ref[:, 0:1], 1.0)
        g0 = ln(s_ref[...] / cnt, dgg[...], dgb[...])
        v = jnp.dot(g0, wv[...], preferred_element_type=jnp.float32) + bv[...]
        g = jnp.dot(v, wo[...], preferred_element_type=jnp.float32) + bo[...]
        res = g
        h = ln(g, fng[...], fnb[...])
        h = jnp.maximum(jnp.dot(h, w1[...], preferred_element_type=jnp.float32) + b1[...], 0.0)
        g = jnp.dot(h, w2[...], preferred_element_type=jnp.float32) + b2[...] + res
        gp_ref[...] = jnp.dot(g, whi[...], preferred_element_type=jnp.float32) + fb[...]

    r = lambda a: a.reshape(1, -1)
    return pl.pallas_call(
        body,
        out_shape=jax.ShapeDtypeStruct((P, 256), jnp.float32),
    )(sums, counts, r(p['dg_norm_g']), r(p['dg_norm_b']),
      p['Wv'], r(p['bv']), p['Wo'], r(p['bo']),
      r(p['ffn_norm_g']), r(p['ffn_norm_b']),
      p['ffn_W1'], r(p['ffn_b1']), p['ffn_W2'], r(p['ffn_b2']),
      p['fuse_W'][256:], r(p['fuse_b']))


def _tc_fuse3(xn, gp, wlo, batchT, clsW, clsb, P):
    n = xn.shape[0]
    nb = n // BN
    d_out = clsW.shape[1]

    def body(xn_ref, gp_ref, w_ref, bt_ref, cw_ref, cb_ref, fused_ref, out_ref):
        iot = lax.broadcasted_iota(jnp.int32, (BN, P), 1)
        oh = (iot == jnp.broadcast_to(bt_ref[...], (BN, P))).astype(jnp.float32)
        pn = jnp.dot(oh, gp_ref[...], preferred_element_type=jnp.float32)
        z = jnp.dot(xn_ref[...], w_ref[...], preferred_element_type=jnp.float32) + pn
        fused = jnp.maximum(z, 0.0) + xn_ref[...]
        fused_ref[...] = fused
        out_ref[...] = jnp.dot(fused, cw_ref[...], preferred_element_type=jnp.float32) + cb_ref[...]

    return pl.pallas_call(
        body,
        grid=(nb,),
        in_specs=[
            pl.BlockSpec((BN, 256), lambda i: (i, 0)),
            pl.BlockSpec((P, 256), lambda i: (0, 0)),
            pl.BlockSpec((256, 256), lambda i: (0, 0)),
            pl.BlockSpec((BN, 1), lambda i: (i, 0)),
            pl.BlockSpec((256, d_out), lambda i: (0, 0)),
            pl.BlockSpec((1, d_out), lambda i: (0, 0)),
        ],
        out_specs=[
            pl.BlockSpec((BN, 256), lambda i: (i, 0)),
            pl.BlockSpec((BN, d_out), lambda i: (i, 0)),
        ],
        out_shape=[
            jax.ShapeDtypeStruct((n, 256), jnp.float32),
            jax.ShapeDtypeStruct((n, d_out), jnp.float32),
        ],
    )(xn, gp, wlo, batchT, clsW, clsb.reshape(1, d_out))


def kernel(x, original_edge_index, dg_edge_index, batch, params):
    p = params
    n = x.shape[0]
    e = original_edge_index.shape[1]
    R = e // LANE
    P = 64

    src_o, dst_o = original_edge_index[0], original_edge_index[1]
    src_d, dst_d = dg_edge_index[0], dg_edge_index[1]
    srcb_o = jnp.stack([src_o, src_o + n]).reshape(2, R, LANE)
    srcb_d = jnp.stack([src_d, src_d + n]).reshape(2, R, LANE)
    dstr_o = dst_o.reshape(R, LANE)
    dstr_d = dst_d.reshape(R, LANE)
    dstb = jnp.stack([dstr_o, dstr_d])
    npad = -(-n // LANE) * LANE
    zeros128 = jnp.zeros((npad // NTILE, 128), jnp.float32)
    zeros16 = jnp.zeros((npad // NTILE, 16), jnp.float32)
    ones16 = jnp.ones((LANE, 16), jnp.float32)

    hist = _sc_degree(dstb, zeros16, ones16, n)

    dinv_o, hp_o = _tc_prep(x, hist[0], p['o_W1'], p['o_b1'])
    dinv_d, hp_d = _tc_prep(x, hist[1], p['d_W1'], p['d_b1'])

    asum1_o = _sc_conv(hp_o.reshape(2 * n, 128), srcb_o, dstr_o, zeros128, n)
    asum1_d = _sc_conv(hp_d.reshape(2 * n, 128), srcb_d, dstr_d, zeros128, n)

    hp2_o = _tc_mid(asum1_o, hp_o, dinv_o, p['o_W2'], p['o_b2'])
    hp2_d = _tc_mid(asum1_d, hp_d, dinv_d, p['d_W2'], p['d_b2'])

    asum2_o = _sc_conv(hp2_o.reshape(2 * n, 128), srcb_o, dstr_o, zeros128, n)
    asum2_d = _sc_conv(hp2_d.reshape(2 * n, 128), srcb_d, dstr_d, zeros128, n)

    batch3 = batch.reshape(n // BN, 1, BN)
    orig, dg, xn, sums, counts = _tc_fuse1(
        asum2_o, hp2_o, dinv_o, asum2_d, hp2_d, dinv_d, batch3,
        p['node_norm_g'], p['node_norm_b'], P)
    gp = _tc_fuse2(sums, counts, p)
    fused, out = _tc_fuse3(xn, gp, p['fuse_W'][:256], batch.reshape(n, 1),
                           p['cls_W'], p['cls_b'], P)
    return (out, orig, dg, fused)
